# baseline (speedup 1.0000x reference)
import jax
import jax.numpy as jnp
from jax.experimental import pallas as pl

_BB = 256


def _body(x_ref, m_ref, o_ref):
    d = x_ref.shape[1] // 2
    o_ref[:, :d] = x_ref[:, :d] * m_ref[:, 0:1]
    o_ref[:, d:] = x_ref[:, d:] * m_ref[:, 1:2]


def kernel(x, head_tail_mask):
    b, s, d = x.shape
    m = head_tail_mask.astype(x.dtype)
    x2 = x.reshape(b, s * d)
    return pl.pallas_call(
        _body,
        grid=(b // _BB,),
        in_specs=[
            pl.BlockSpec((_BB, s * d), lambda i: (i, 0)),
            pl.BlockSpec((_BB, s), lambda i: (i, 0)),
        ],
        out_specs=pl.BlockSpec((_BB, s * d), lambda i: (i, 0)),
        out_shape=jax.ShapeDtypeStruct((b, s * d), x.dtype),
    )(x2, m)


# speedup vs baseline: 1.0405x; 1.0405x over previous
import jax
import jax.numpy as jnp
from jax.experimental import pallas as pl

_BB = 512


def _body(x_ref, m_ref, o_ref):
    d = x_ref.shape[1] // 2
    o_ref[:, :d] = x_ref[:, :d] * m_ref[:, 0:1]
    o_ref[:, d:] = x_ref[:, d:] * m_ref[:, 1:2]


def kernel(x, head_tail_mask):
    b, s, d = x.shape
    m = head_tail_mask.astype(x.dtype)
    x2 = x.reshape(b, s * d)
    return pl.pallas_call(
        _body,
        grid=(b // _BB,),
        in_specs=[
            pl.BlockSpec((_BB, s * d), lambda i: (i, 0)),
            pl.BlockSpec((_BB, s), lambda i: (i, 0)),
        ],
        out_specs=pl.BlockSpec((_BB, s * d), lambda i: (i, 0)),
        out_shape=jax.ShapeDtypeStruct((b, s * d), x.dtype),
    )(x2, m)


# speedup vs baseline: 1.0458x; 1.0050x over previous
import jax
import jax.numpy as jnp
from jax.experimental import pallas as pl

_BB = 1024


def _body(x_ref, m_ref, o_ref):
    d = x_ref.shape[1] // 2
    o_ref[:, :d] = x_ref[:, :d] * m_ref[:, 0:1]
    o_ref[:, d:] = x_ref[:, d:] * m_ref[:, 1:2]


def kernel(x, head_tail_mask):
    b, s, d = x.shape
    m = head_tail_mask.astype(x.dtype)
    x2 = x.reshape(b, s * d)
    return pl.pallas_call(
        _body,
        grid=(b // _BB,),
        in_specs=[
            pl.BlockSpec((_BB, s * d), lambda i: (i, 0)),
            pl.BlockSpec((_BB, s), lambda i: (i, 0)),
        ],
        out_specs=pl.BlockSpec((_BB, s * d), lambda i: (i, 0)),
        out_shape=jax.ShapeDtypeStruct((b, s * d), x.dtype),
    )(x2, m)
